# Initial kernel scaffold; baseline (speedup 1.0000x reference)
import jax
import jax.numpy as jnp
from jax.experimental import pallas as pl


def kernel(x, edge_index, W_in, b_in, W1, b1, W2, b2):
    raise NotImplementedError("write your pallas kernel here")


# speedup vs baseline: 14.3515x; 14.3515x over previous
import functools

import jax
import jax.numpy as jnp
from jax import lax
from jax.experimental import pallas as pl
from jax.experimental.pallas import tpu as pltpu
from jax.experimental.pallas import tpu_sc as plsc

N = 10000
E = 320000
D = 128

NC = 2
NS = 16
NW = NC * NS
CH = 128
NCHUNK = E // CH
ITERS = -(-NCHUNK // NW)
ROWS_PT = 624
ROWS_LAST = N - (NS - 1) * ROWS_PT


def _rowwise(copy_fn, sid):
  base = pl.multiple_of(sid * ROWS_PT, 8)

  @pl.when(sid < NS - 1)
  def _():
    copy_fn(base, ROWS_PT)

  @pl.when(sid == NS - 1)
  def _():
    copy_fn(base, ROWS_LAST)

BN = 400

@functools.cache
def _mesh():
  return plsc.VectorSubcoreMesh(core_axis_name="c", subcore_axis_name="s",
                                num_cores=NC, num_subcores=NS)


def _sc_degree(dst, ones_nd, zeros_nd):
  @functools.partial(
      pl.kernel,
      out_type=jax.ShapeDtypeStruct((NC * N, D), jnp.float32),
      mesh=_mesh(),
      scratch_types=[
          pltpu.VMEM((CH,), jnp.int32),
          pltpu.VMEM((CH, D), jnp.float32),
          pltpu.VMEM_SHARED((N, D), jnp.float32),
      ],
  )
  def k(dst_hbm, ones_hbm, zeros_hbm, out_hbm, idxv, onesv, acc):
    cid = lax.axis_index("c")
    sid = lax.axis_index("s")
    w = cid * NS + sid

    def init(start, nrows):
      pltpu.sync_copy(zeros_hbm.at[pl.ds(start, nrows)],
                      acc.at[pl.ds(start, nrows)])

    _rowwise(init, sid)
    pltpu.sync_copy(ones_hbm, onesv)
    plsc.subcore_barrier()

    @pl.loop(0, ITERS)
    def _(i):
      c = w + i * NW

      @pl.when(c < NCHUNK)
      def _():
        pltpu.sync_copy(dst_hbm.at[pl.ds(c * CH, CH)], idxv)
        pltpu.sync_copy(onesv, acc.at[idxv], add=True)

    plsc.subcore_barrier()

    def writeback(start, nrows):
      pltpu.sync_copy(acc.at[pl.ds(start, nrows)],
                      out_hbm.at[pl.ds(pl.multiple_of(cid * N + start, 8),
                                       nrows)])

    _rowwise(writeback, sid)

  return k(dst, ones_nd, zeros_nd)


def _sc_scatter(g, src, dst, zeros):
  @functools.partial(
      pl.kernel,
      out_type=jax.ShapeDtypeStruct((NC * N, D), jnp.float32),
      mesh=_mesh(),
      scratch_types=[
          pltpu.VMEM((CH,), jnp.int32),
          pltpu.VMEM((CH,), jnp.int32),
          pltpu.VMEM((CH, D), jnp.float32),
          pltpu.VMEM_SHARED((N, D), jnp.float32),
      ],
  )
  def k(g_hbm, src_hbm, dst_hbm, zeros_hbm, out_hbm, srcv, dstv, rows, acc):
    cid = lax.axis_index("c")
    sid = lax.axis_index("s")
    w = cid * NS + sid

    def init(start, nrows):
      s = pl.ds(start, nrows)

      @pl.when(cid == 0)
      def _():
        pltpu.sync_copy(g_hbm.at[s], acc.at[s])

      @pl.when(cid != 0)
      def _():
        pltpu.sync_copy(zeros_hbm.at[s], acc.at[s])

    _rowwise(init, sid)
    plsc.subcore_barrier()

    @pl.loop(0, ITERS)
    def _(i):
      c = w + i * NW

      @pl.when(c < NCHUNK)
      def _():
        pltpu.sync_copy(src_hbm.at[pl.ds(c * CH, CH)], srcv)
        pltpu.sync_copy(dst_hbm.at[pl.ds(c * CH, CH)], dstv)
        pltpu.sync_copy(g_hbm.at[srcv], rows)
        pltpu.sync_copy(rows, acc.at[dstv], add=True)

    plsc.subcore_barrier()

    def writeback(start, nrows):
      pltpu.sync_copy(acc.at[pl.ds(start, nrows)],
                      out_hbm.at[pl.ds(pl.multiple_of(cid * N + start, 8),
                                       nrows)])

    _rowwise(writeback, sid)

  return k(g, src, dst, zeros)


_DOT = functools.partial(jnp.dot, preferred_element_type=jnp.float32,
                         precision=lax.Precision.HIGHEST)


def _tc_call(body, n_in, extra_specs):
  return pl.pallas_call(
      body,
      grid=(N // BN,),
      in_specs=extra_specs,
      out_specs=pl.BlockSpec((BN, D), lambda i: (i, 0)),
      out_shape=jax.ShapeDtypeStruct((N, D), jnp.float32),
  )


def _linear(x, WT, b):
  def body(x_ref, w_ref, b_ref, o_ref):
    o_ref[...] = _DOT(x_ref[...], w_ref[...]) + b_ref[...]

  return pl.pallas_call(
      body,
      grid=(N // BN,),
      in_specs=[pl.BlockSpec((BN, D), lambda i: (i, 0)),
                pl.BlockSpec((D, D), lambda i: (0, 0)),
                pl.BlockSpec((1, D), lambda i: (0, 0))],
      out_specs=pl.BlockSpec((BN, D), lambda i: (i, 0)),
      out_shape=jax.ShapeDtypeStruct((N, D), jnp.float32),
  )(x, WT, b.reshape(1, D))


def _linear_scaled(h, WT, dinv):
  def body(h_ref, w_ref, s_ref, o_ref):
    o_ref[...] = _DOT(h_ref[...], w_ref[...]) * s_ref[...]

  return pl.pallas_call(
      body,
      grid=(N // BN,),
      in_specs=[pl.BlockSpec((BN, D), lambda i: (i, 0)),
                pl.BlockSpec((D, D), lambda i: (0, 0)),
                pl.BlockSpec((BN, 1), lambda i: (i, 0))],
      out_specs=pl.BlockSpec((BN, D), lambda i: (i, 0)),
      out_shape=jax.ShapeDtypeStruct((N, D), jnp.float32),
  )(h, WT, dinv)


def _combine_linear(acc, dinv, b, WT):
  def body(a0_ref, a1_ref, s_ref, b_ref, w_ref, o_ref):
    h = jnp.maximum((a0_ref[...] + a1_ref[...]) * s_ref[...] + b_ref[...], 0.0)
    o_ref[...] = _DOT(h, w_ref[...]) * s_ref[...]

  nb = N // BN
  return pl.pallas_call(
      body,
      grid=(nb,),
      in_specs=[pl.BlockSpec((BN, D), lambda i: (i, 0)),
                pl.BlockSpec((BN, D), lambda i: (i + nb, 0)),
                pl.BlockSpec((BN, 1), lambda i: (i, 0)),
                pl.BlockSpec((1, D), lambda i: (0, 0)),
                pl.BlockSpec((D, D), lambda i: (0, 0))],
      out_specs=pl.BlockSpec((BN, D), lambda i: (i, 0)),
      out_shape=jax.ShapeDtypeStruct((N, D), jnp.float32),
  )(acc, acc, dinv, b.reshape(1, D), WT)


def _finalize(acc, dinv, b):
  def body(a0_ref, a1_ref, s_ref, b_ref, o_ref):
    o_ref[...] = jnp.maximum(
        (a0_ref[...] + a1_ref[...]) * s_ref[...] + b_ref[...], 0.0)

  nb = N // BN
  return pl.pallas_call(
      body,
      grid=(nb,),
      in_specs=[pl.BlockSpec((BN, D), lambda i: (i, 0)),
                pl.BlockSpec((BN, D), lambda i: (i + nb, 0)),
                pl.BlockSpec((BN, 1), lambda i: (i, 0)),
                pl.BlockSpec((1, D), lambda i: (0, 0))],
      out_specs=pl.BlockSpec((BN, D), lambda i: (i, 0)),
      out_shape=jax.ShapeDtypeStruct((N, D), jnp.float32),
  )(acc, acc, dinv, b.reshape(1, D))


def kernel(x, edge_index, W_in, b_in, W1, b1, W2, b2):
  src = edge_index[0]
  dst = edge_index[1]
  zeros_nd = jnp.zeros((N, D), jnp.float32)
  ones_ch = jnp.ones((CH, D), jnp.float32)

  deg2 = _sc_degree(dst, ones_ch, zeros_nd)
  h0 = _linear(x, W_in.T, b_in)
  deg = deg2[:N, 0] + deg2[N:, 0] + 1.0
  dinv = lax.rsqrt(deg).reshape(N, 1)

  g1 = _linear_scaled(h0, W1.T, dinv)
  acc1 = _sc_scatter(g1, src, dst, zeros_nd)
  g2 = _combine_linear(acc1, dinv, b1, W2.T)
  acc2 = _sc_scatter(g2, src, dst, zeros_nd)
  return _finalize(acc2, dinv, b2)


# speedup vs baseline: 26.0418x; 1.8146x over previous
import functools

import jax
import jax.numpy as jnp
from jax import lax
from jax.experimental import pallas as pl
from jax.experimental.pallas import tpu as pltpu
from jax.experimental.pallas import tpu_sc as plsc

N = 10000
E = 320000
D = 128

NC = 2
NS = 16
NW = NC * NS
CH = 128
NCHUNK = E // CH
ITERS = -(-NCHUNK // NW)
ROWS_PT = 624
ROWS_LAST = N - (NS - 1) * ROWS_PT


def _rowwise(copy_fn, sid):
  base = pl.multiple_of(sid * ROWS_PT, 8)

  @pl.when(sid < NS - 1)
  def _():
    copy_fn(base, ROWS_PT)

  @pl.when(sid == NS - 1)
  def _():
    copy_fn(base, ROWS_LAST)

BN = 400

@functools.cache
def _mesh():
  return plsc.VectorSubcoreMesh(core_axis_name="c", subcore_axis_name="s",
                                num_cores=NC, num_subcores=NS)


def _sc_degree(dst, ones_nd, zeros_nd):
  @functools.partial(
      pl.kernel,
      out_type=jax.ShapeDtypeStruct((NC * N, D), jnp.float32),
      mesh=_mesh(),
      scratch_types=[
          pltpu.VMEM((2, CH), jnp.int32),
          pltpu.VMEM((CH, D), jnp.float32),
          pltpu.VMEM_SHARED((N, D), jnp.float32),
          pltpu.SemaphoreType.DMA((2,)),
      ],
  )
  def k(dst_hbm, ones_hbm, zeros_hbm, out_hbm, idxv, onesv, acc, sidx):
    cid = lax.axis_index("c")
    sid = lax.axis_index("s")
    w = cid * NS + sid
    ti = (NCHUNK - 1 - w) // NW + 1

    def init(start, nrows):
      pltpu.sync_copy(zeros_hbm.at[pl.ds(start, nrows)],
                      acc.at[pl.ds(start, nrows)])

    _rowwise(init, sid)
    pltpu.sync_copy(ones_hbm, onesv)

    def eoff(i):
      return pl.ds(pl.multiple_of((w + i * NW) * CH, 8), CH)

    def idx_copy(i):
      ib = lax.rem(i, 2)
      return pltpu.make_async_copy(dst_hbm.at[eoff(i)], idxv.at[ib],
                                   sidx.at[ib])

    idx_copy(0).start()
    plsc.subcore_barrier()

    @pl.loop(0, ITERS)
    def _(i):
      @pl.when(i < ti)
      def _():
        idx_copy(i).wait()

        @pl.when(i + 1 < ti)
        def _():
          idx_copy(i + 1).start()

        pltpu.sync_copy(onesv, acc.at[idxv.at[lax.rem(i, 2)]], add=True)

    plsc.subcore_barrier()

    def writeback(start, nrows):
      pltpu.sync_copy(acc.at[pl.ds(start, nrows)],
                      out_hbm.at[pl.ds(pl.multiple_of(cid * N + start, 8),
                                       nrows)])

    _rowwise(writeback, sid)

  return k(dst, ones_nd, zeros_nd)


def _sc_scatter(g, src, dst, zeros):
  @functools.partial(
      pl.kernel,
      out_type=jax.ShapeDtypeStruct((NC * N, D), jnp.float32),
      mesh=_mesh(),
      scratch_types=[
          pltpu.VMEM((3, CH), jnp.int32),
          pltpu.VMEM((3, CH), jnp.int32),
          pltpu.VMEM((2, CH, D), jnp.float32),
          pltpu.VMEM_SHARED((N, D), jnp.float32),
          pltpu.SemaphoreType.DMA((3,)),
          pltpu.SemaphoreType.DMA((2,)),
      ],
  )
  def k(g_hbm, src_hbm, dst_hbm, zeros_hbm, out_hbm, srcv, dstv, rows, acc,
        sidx, sgat):
    cid = lax.axis_index("c")
    sid = lax.axis_index("s")
    w = cid * NS + sid
    ti = (NCHUNK - 1 - w) // NW + 1

    def init(start, nrows):
      s = pl.ds(start, nrows)

      @pl.when(cid == 0)
      def _():
        pltpu.sync_copy(g_hbm.at[s], acc.at[s])

      @pl.when(cid != 0)
      def _():
        pltpu.sync_copy(zeros_hbm.at[s], acc.at[s])

    _rowwise(init, sid)

    def eoff(i):
      return pl.ds(pl.multiple_of((w + i * NW) * CH, 8), CH)

    def idx_start(i):
      ib = lax.rem(i, 3)
      pltpu.make_async_copy(src_hbm.at[eoff(i)], srcv.at[ib],
                            sidx.at[ib]).start()
      pltpu.make_async_copy(dst_hbm.at[eoff(i)], dstv.at[ib],
                            sidx.at[ib]).start()

    def idx_wait(i):
      ib = lax.rem(i, 3)
      pltpu.make_async_copy(src_hbm.at[eoff(i)], srcv.at[ib],
                            sidx.at[ib]).wait()
      pltpu.make_async_copy(dst_hbm.at[eoff(i)], dstv.at[ib],
                            sidx.at[ib]).wait()

    def gather_start(i):
      ib = lax.rem(i, 3)
      rb = lax.rem(i, 2)
      pltpu.make_async_copy(g_hbm.at[srcv.at[ib]], rows.at[rb],
                            sgat.at[rb]).start()

    def gather_wait(i):
      ib = lax.rem(i, 3)
      rb = lax.rem(i, 2)
      pltpu.make_async_copy(g_hbm.at[srcv.at[ib]], rows.at[rb],
                            sgat.at[rb]).wait()

    idx_start(0)
    idx_wait(0)
    gather_start(0)

    @pl.when(1 < ti)
    def _():
      idx_start(1)

    plsc.subcore_barrier()

    @pl.loop(0, ITERS)
    def _(i):
      @pl.when(i < ti)
      def _():
        @pl.when(i + 1 < ti)
        def _():
          idx_wait(i + 1)
          gather_start(i + 1)

        gather_wait(i)

        @pl.when(i + 2 < ti)
        def _():
          idx_start(i + 2)

        pltpu.sync_copy(rows.at[lax.rem(i, 2)],
                        acc.at[dstv.at[lax.rem(i, 3)]], add=True)

    plsc.subcore_barrier()

    def writeback(start, nrows):
      pltpu.sync_copy(acc.at[pl.ds(start, nrows)],
                      out_hbm.at[pl.ds(pl.multiple_of(cid * N + start, 8),
                                       nrows)])

    _rowwise(writeback, sid)

  return k(g, src, dst, zeros)


_DOT = functools.partial(jnp.dot, preferred_element_type=jnp.float32,
                         precision=lax.Precision.HIGHEST)


def _tc_call(body, n_in, extra_specs):
  return pl.pallas_call(
      body,
      grid=(N // BN,),
      in_specs=extra_specs,
      out_specs=pl.BlockSpec((BN, D), lambda i: (i, 0)),
      out_shape=jax.ShapeDtypeStruct((N, D), jnp.float32),
  )


def _linear(x, WT, b):
  def body(x_ref, w_ref, b_ref, o_ref):
    o_ref[...] = _DOT(x_ref[...], w_ref[...]) + b_ref[...]

  return pl.pallas_call(
      body,
      grid=(N // BN,),
      in_specs=[pl.BlockSpec((BN, D), lambda i: (i, 0)),
                pl.BlockSpec((D, D), lambda i: (0, 0)),
                pl.BlockSpec((1, D), lambda i: (0, 0))],
      out_specs=pl.BlockSpec((BN, D), lambda i: (i, 0)),
      out_shape=jax.ShapeDtypeStruct((N, D), jnp.float32),
  )(x, WT, b.reshape(1, D))


def _linear_scaled(h, WT, dinv):
  def body(h_ref, w_ref, s_ref, o_ref):
    o_ref[...] = _DOT(h_ref[...], w_ref[...]) * s_ref[...]

  return pl.pallas_call(
      body,
      grid=(N // BN,),
      in_specs=[pl.BlockSpec((BN, D), lambda i: (i, 0)),
                pl.BlockSpec((D, D), lambda i: (0, 0)),
                pl.BlockSpec((BN, 1), lambda i: (i, 0))],
      out_specs=pl.BlockSpec((BN, D), lambda i: (i, 0)),
      out_shape=jax.ShapeDtypeStruct((N, D), jnp.float32),
  )(h, WT, dinv)


def _combine_linear(acc, dinv, b, WT):
  def body(a0_ref, a1_ref, s_ref, b_ref, w_ref, o_ref):
    h = jnp.maximum((a0_ref[...] + a1_ref[...]) * s_ref[...] + b_ref[...], 0.0)
    o_ref[...] = _DOT(h, w_ref[...]) * s_ref[...]

  nb = N // BN
  return pl.pallas_call(
      body,
      grid=(nb,),
      in_specs=[pl.BlockSpec((BN, D), lambda i: (i, 0)),
                pl.BlockSpec((BN, D), lambda i: (i + nb, 0)),
                pl.BlockSpec((BN, 1), lambda i: (i, 0)),
                pl.BlockSpec((1, D), lambda i: (0, 0)),
                pl.BlockSpec((D, D), lambda i: (0, 0))],
      out_specs=pl.BlockSpec((BN, D), lambda i: (i, 0)),
      out_shape=jax.ShapeDtypeStruct((N, D), jnp.float32),
  )(acc, acc, dinv, b.reshape(1, D), WT)


def _finalize(acc, dinv, b):
  def body(a0_ref, a1_ref, s_ref, b_ref, o_ref):
    o_ref[...] = jnp.maximum(
        (a0_ref[...] + a1_ref[...]) * s_ref[...] + b_ref[...], 0.0)

  nb = N // BN
  return pl.pallas_call(
      body,
      grid=(nb,),
      in_specs=[pl.BlockSpec((BN, D), lambda i: (i, 0)),
                pl.BlockSpec((BN, D), lambda i: (i + nb, 0)),
                pl.BlockSpec((BN, 1), lambda i: (i, 0)),
                pl.BlockSpec((1, D), lambda i: (0, 0))],
      out_specs=pl.BlockSpec((BN, D), lambda i: (i, 0)),
      out_shape=jax.ShapeDtypeStruct((N, D), jnp.float32),
  )(acc, acc, dinv, b.reshape(1, D))


def kernel(x, edge_index, W_in, b_in, W1, b1, W2, b2):
  src = edge_index[0]
  dst = edge_index[1]
  zeros_nd = jnp.zeros((N, D), jnp.float32)
  ones_ch = jnp.ones((CH, D), jnp.float32)

  deg2 = _sc_degree(dst, ones_ch, zeros_nd)
  h0 = _linear(x, W_in.T, b_in)
  deg = deg2[:N, 0] + deg2[N:, 0] + 1.0
  dinv = lax.rsqrt(deg).reshape(N, 1)

  g1 = _linear_scaled(h0, W1.T, dinv)
  acc1 = _sc_scatter(g1, src, dst, zeros_nd)
  g2 = _combine_linear(acc1, dinv, b1, W2.T)
  acc2 = _sc_scatter(g2, src, dst, zeros_nd)
  return _finalize(acc2, dinv, b2)


# speedup vs baseline: 28.0677x; 1.0778x over previous
import dataclasses
import functools

import jax
import jax.numpy as jnp
from jax import lax
from jax.experimental import pallas as pl
from jax.experimental.pallas import tpu as pltpu
from jax.experimental.pallas import tpu_sc as plsc

N = 10000
E = 320000
D = 128

NC = 2
NS = 16
NW = NC * NS
CH = 128
NCHUNK = E // CH
ITERS = -(-NCHUNK // NW)
ROWS_PT = 624
ROWS_LAST = N - (NS - 1) * ROWS_PT


def _rowwise(copy_fn, sid):
  base = pl.multiple_of(sid * ROWS_PT, 8)

  @pl.when(sid < NS - 1)
  def _():
    copy_fn(base, ROWS_PT)

  @pl.when(sid == NS - 1)
  def _():
    copy_fn(base, ROWS_LAST)

BN = 400

def _sc_compiler_params():
  cp = pltpu.CompilerParams()
  if "needs_layout_passes" in pltpu.CompilerParams.__dataclass_fields__:
    cp = dataclasses.replace(cp, needs_layout_passes=False)
  return cp


@functools.cache
def _mesh():
  return plsc.VectorSubcoreMesh(core_axis_name="c", subcore_axis_name="s",
                                num_cores=NC, num_subcores=NS)


def _sc_degree(dst):
  @functools.partial(
      pl.kernel,
      out_type=jax.ShapeDtypeStruct((NW * N,), jnp.float32),
      mesh=_mesh(),
      scratch_types=[
          pltpu.VMEM((2 * CH,), jnp.int32),
          pltpu.VMEM((N,), jnp.float32),
          pltpu.SemaphoreType.DMA((2,)),
      ],
      compiler_params=_sc_compiler_params(),
  )
  def k(dst_hbm, out_hbm, idxv, hist, sidx):
    cid = lax.axis_index("c")
    sid = lax.axis_index("s")
    w = cid * NS + sid
    ti = (NCHUNK - 1 - w) // NW + 1

    @pl.loop(0, N // 16)
    def _(i):
      hist[pl.ds(i * 16, 16)] = jnp.zeros((16,), jnp.float32)

    def eoff(i):
      return pl.ds(pl.multiple_of((w + i * NW) * CH, 8), CH)

    def idx_copy(i):
      ib = lax.rem(i, 2)
      return pltpu.make_async_copy(dst_hbm.at[eoff(i)],
                                   idxv.at[pl.ds(ib * CH, CH)], sidx.at[ib])

    idx_copy(0).start()

    @pl.loop(0, ITERS)
    def _(i):
      @pl.when(i < ti)
      def _():
        idx_copy(i).wait()

        @pl.when(i + 1 < ti)
        def _():
          idx_copy(i + 1).start()

        base = lax.rem(i, 2) * CH
        for j in range(CH // 16):
          v = idxv[pl.ds(base + j * 16, 16)]
          plsc.addupdate_scatter(hist, [v], jnp.ones((16,), jnp.float32))

    pltpu.sync_copy(hist, out_hbm.at[pl.ds(pl.multiple_of(w * N, 8), N)])

  return k(dst)


def _sc_scatter(g, src, dst, zeros):
  @functools.partial(
      pl.kernel,
      out_type=jax.ShapeDtypeStruct((NC * N, D), jnp.float32),
      mesh=_mesh(),
      scratch_types=[
          pltpu.VMEM((3, CH), jnp.int32),
          pltpu.VMEM((3, CH), jnp.int32),
          pltpu.VMEM((2, CH, D), jnp.float32),
          pltpu.VMEM_SHARED((N, D), jnp.float32),
          pltpu.SemaphoreType.DMA((3,)),
          pltpu.SemaphoreType.DMA((2,)),
      ],
  )
  def k(g_hbm, src_hbm, dst_hbm, zeros_hbm, out_hbm, srcv, dstv, rows, acc,
        sidx, sgat):
    cid = lax.axis_index("c")
    sid = lax.axis_index("s")
    w = cid * NS + sid
    ti = (NCHUNK - 1 - w) // NW + 1

    def init(start, nrows):
      s = pl.ds(start, nrows)

      @pl.when(cid == 0)
      def _():
        pltpu.sync_copy(g_hbm.at[s], acc.at[s])

      @pl.when(cid != 0)
      def _():
        pltpu.sync_copy(zeros_hbm.at[s], acc.at[s])

    _rowwise(init, sid)

    def eoff(i):
      return pl.ds(pl.multiple_of((w + i * NW) * CH, 8), CH)

    def idx_start(i):
      ib = lax.rem(i, 3)
      pltpu.make_async_copy(src_hbm.at[eoff(i)], srcv.at[ib],
                            sidx.at[ib]).start()
      pltpu.make_async_copy(dst_hbm.at[eoff(i)], dstv.at[ib],
                            sidx.at[ib]).start()

    def idx_wait(i):
      ib = lax.rem(i, 3)
      pltpu.make_async_copy(src_hbm.at[eoff(i)], srcv.at[ib],
                            sidx.at[ib]).wait()
      pltpu.make_async_copy(dst_hbm.at[eoff(i)], dstv.at[ib],
                            sidx.at[ib]).wait()

    def gather_start(i):
      ib = lax.rem(i, 3)
      rb = lax.rem(i, 2)
      pltpu.make_async_copy(g_hbm.at[srcv.at[ib]], rows.at[rb],
                            sgat.at[rb]).start()

    def gather_wait(i):
      ib = lax.rem(i, 3)
      rb = lax.rem(i, 2)
      pltpu.make_async_copy(g_hbm.at[srcv.at[ib]], rows.at[rb],
                            sgat.at[rb]).wait()

    idx_start(0)
    idx_wait(0)
    gather_start(0)

    @pl.when(1 < ti)
    def _():
      idx_start(1)

    plsc.subcore_barrier()

    @pl.loop(0, ITERS)
    def _(i):
      @pl.when(i < ti)
      def _():
        @pl.when(i + 1 < ti)
        def _():
          idx_wait(i + 1)
          gather_start(i + 1)

        gather_wait(i)

        @pl.when(i + 2 < ti)
        def _():
          idx_start(i + 2)

        pltpu.sync_copy(rows.at[lax.rem(i, 2)],
                        acc.at[dstv.at[lax.rem(i, 3)]], add=True)

    plsc.subcore_barrier()

    def writeback(start, nrows):
      pltpu.sync_copy(acc.at[pl.ds(start, nrows)],
                      out_hbm.at[pl.ds(pl.multiple_of(cid * N + start, 8),
                                       nrows)])

    _rowwise(writeback, sid)

  return k(g, src, dst, zeros)


_DOT = functools.partial(jnp.dot, preferred_element_type=jnp.float32,
                         precision=lax.Precision.HIGHEST)


def _tc_call(body, n_in, extra_specs):
  return pl.pallas_call(
      body,
      grid=(N // BN,),
      in_specs=extra_specs,
      out_specs=pl.BlockSpec((BN, D), lambda i: (i, 0)),
      out_shape=jax.ShapeDtypeStruct((N, D), jnp.float32),
  )


def _linear(x, WT, b):
  def body(x_ref, w_ref, b_ref, o_ref):
    o_ref[...] = _DOT(x_ref[...], w_ref[...]) + b_ref[...]

  return pl.pallas_call(
      body,
      grid=(N // BN,),
      in_specs=[pl.BlockSpec((BN, D), lambda i: (i, 0)),
                pl.BlockSpec((D, D), lambda i: (0, 0)),
                pl.BlockSpec((1, D), lambda i: (0, 0))],
      out_specs=pl.BlockSpec((BN, D), lambda i: (i, 0)),
      out_shape=jax.ShapeDtypeStruct((N, D), jnp.float32),
  )(x, WT, b.reshape(1, D))


def _linear_scaled(h, WT, dinv):
  def body(h_ref, w_ref, s_ref, o_ref):
    o_ref[...] = _DOT(h_ref[...], w_ref[...]) * s_ref[...]

  return pl.pallas_call(
      body,
      grid=(N // BN,),
      in_specs=[pl.BlockSpec((BN, D), lambda i: (i, 0)),
                pl.BlockSpec((D, D), lambda i: (0, 0)),
                pl.BlockSpec((BN, 1), lambda i: (i, 0))],
      out_specs=pl.BlockSpec((BN, D), lambda i: (i, 0)),
      out_shape=jax.ShapeDtypeStruct((N, D), jnp.float32),
  )(h, WT, dinv)


def _combine_linear(acc, dinv, b, WT):
  def body(a0_ref, a1_ref, s_ref, b_ref, w_ref, o_ref):
    h = jnp.maximum((a0_ref[...] + a1_ref[...]) * s_ref[...] + b_ref[...], 0.0)
    o_ref[...] = _DOT(h, w_ref[...]) * s_ref[...]

  nb = N // BN
  return pl.pallas_call(
      body,
      grid=(nb,),
      in_specs=[pl.BlockSpec((BN, D), lambda i: (i, 0)),
                pl.BlockSpec((BN, D), lambda i: (i + nb, 0)),
                pl.BlockSpec((BN, 1), lambda i: (i, 0)),
                pl.BlockSpec((1, D), lambda i: (0, 0)),
                pl.BlockSpec((D, D), lambda i: (0, 0))],
      out_specs=pl.BlockSpec((BN, D), lambda i: (i, 0)),
      out_shape=jax.ShapeDtypeStruct((N, D), jnp.float32),
  )(acc, acc, dinv, b.reshape(1, D), WT)


def _finalize(acc, dinv, b):
  def body(a0_ref, a1_ref, s_ref, b_ref, o_ref):
    o_ref[...] = jnp.maximum(
        (a0_ref[...] + a1_ref[...]) * s_ref[...] + b_ref[...], 0.0)

  nb = N // BN
  return pl.pallas_call(
      body,
      grid=(nb,),
      in_specs=[pl.BlockSpec((BN, D), lambda i: (i, 0)),
                pl.BlockSpec((BN, D), lambda i: (i + nb, 0)),
                pl.BlockSpec((BN, 1), lambda i: (i, 0)),
                pl.BlockSpec((1, D), lambda i: (0, 0))],
      out_specs=pl.BlockSpec((BN, D), lambda i: (i, 0)),
      out_shape=jax.ShapeDtypeStruct((N, D), jnp.float32),
  )(acc, acc, dinv, b.reshape(1, D))


def kernel(x, edge_index, W_in, b_in, W1, b1, W2, b2):
  src = edge_index[0]
  dst = edge_index[1]
  zeros_nd = jnp.zeros((N, D), jnp.float32)

  deg2 = _sc_degree(dst)
  h0 = _linear(x, W_in.T, b_in)
  deg = deg2.reshape(NW, N).sum(axis=0) + 1.0
  dinv = lax.rsqrt(deg).reshape(N, 1)

  g1 = _linear_scaled(h0, W1.T, dinv)
  acc1 = _sc_scatter(g1, src, dst, zeros_nd)
  g2 = _combine_linear(acc1, dinv, b1, W2.T)
  acc2 = _sc_scatter(g2, src, dst, zeros_nd)
  return _finalize(acc2, dinv, b2)


# speedup vs baseline: 33.5718x; 1.1961x over previous
import dataclasses
import functools

import jax
import jax.numpy as jnp
from jax import lax
from jax.experimental import pallas as pl
from jax.experimental.pallas import tpu as pltpu
from jax.experimental.pallas import tpu_sc as plsc

N = 10000
E = 320000
D = 128

NC = 2
NS = 16
NW = NC * NS
CH = 128
NCHUNK = E // CH
ITERS = -(-NCHUNK // NW)
ROWS_PT = 624
ROWS_LAST = N - (NS - 1) * ROWS_PT

DEG_BLK = 2000
DEG_NB = E // NW // DEG_BLK


def _rowwise(copy_fn, sid):
  base = pl.multiple_of(sid * ROWS_PT, 8)

  @pl.when(sid < NS - 1)
  def _():
    copy_fn(base, ROWS_PT)

  @pl.when(sid == NS - 1)
  def _():
    copy_fn(base, ROWS_LAST)

BN = 400

def _sc_compiler_params():
  cp = pltpu.CompilerParams()
  if "needs_layout_passes" in pltpu.CompilerParams.__dataclass_fields__:
    cp = dataclasses.replace(cp, needs_layout_passes=False)
  return cp


@functools.cache
def _mesh():
  return plsc.VectorSubcoreMesh(core_axis_name="c", subcore_axis_name="s",
                                num_cores=NC, num_subcores=NS)


def _sc_degree(dst):
  @functools.partial(
      pl.kernel,
      out_type=jax.ShapeDtypeStruct((NW * N,), jnp.float32),
      mesh=_mesh(),
      scratch_types=[
          pltpu.VMEM((2 * DEG_BLK,), jnp.int32),
          pltpu.VMEM((N,), jnp.float32),
          pltpu.SemaphoreType.DMA((2,)),
      ],
      compiler_params=_sc_compiler_params(),
  )
  def k(dst_hbm, out_hbm, idxv, hist, sidx):
    cid = lax.axis_index("c")
    sid = lax.axis_index("s")
    w = cid * NS + sid

    @pl.loop(0, N // 16)
    def _(i):
      hist[pl.ds(i * 16, 16)] = jnp.zeros((16,), jnp.float32)

    def idx_copy(i):
      ib = lax.rem(i, 2)
      off = pl.multiple_of(w * (E // NW) + i * DEG_BLK, 8)
      return pltpu.make_async_copy(dst_hbm.at[pl.ds(off, DEG_BLK)],
                                   idxv.at[pl.ds(ib * DEG_BLK, DEG_BLK)],
                                   sidx.at[ib])

    idx_copy(0).start()

    @pl.loop(0, DEG_NB)
    def _(i):
      idx_copy(i).wait()

      @pl.when(i + 1 < DEG_NB)
      def _():
        idx_copy(i + 1).start()

      base = lax.rem(i, 2) * DEG_BLK

      @pl.loop(0, DEG_BLK // 16)
      def _(j):
        v = idxv[pl.ds(base + j * 16, 16)]
        plsc.addupdate_scatter(hist, [v], jnp.ones((16,), jnp.float32))

    pltpu.sync_copy(hist, out_hbm.at[pl.ds(pl.multiple_of(w * N, 8), N)])

  return k(dst)


def _sc_scatter(g, src, dst, zeros):
  @functools.partial(
      pl.kernel,
      out_type=jax.ShapeDtypeStruct((NC * N, D), jnp.float32),
      mesh=_mesh(),
      scratch_types=[
          pltpu.VMEM((4, CH), jnp.int32),
          pltpu.VMEM((4, CH), jnp.int32),
          pltpu.VMEM((3, CH, D), jnp.float32),
          pltpu.VMEM_SHARED((N, D), jnp.float32),
          pltpu.SemaphoreType.DMA((4,)),
          pltpu.SemaphoreType.DMA((3,)),
          pltpu.SemaphoreType.DMA((2,)),
      ],
  )
  def k(g_hbm, src_hbm, dst_hbm, zeros_hbm, out_hbm, srcv, dstv, rows, acc,
        sidx, sgat, sscat):
    cid = lax.axis_index("c")
    sid = lax.axis_index("s")
    w = cid * NS + sid
    ti = (NCHUNK - 1 - w) // NW + 1

    def init(start, nrows):
      s = pl.ds(start, nrows)

      @pl.when(cid == 0)
      def _():
        pltpu.sync_copy(g_hbm.at[s], acc.at[s])

      @pl.when(cid != 0)
      def _():
        pltpu.sync_copy(zeros_hbm.at[s], acc.at[s])

    _rowwise(init, sid)

    def eoff(i):
      return pl.ds(pl.multiple_of((w + i * NW) * CH, 8), CH)

    def idx_copies(i):
      ib = lax.rem(i, 4)
      return (pltpu.make_async_copy(src_hbm.at[eoff(i)], srcv.at[ib],
                                    sidx.at[ib]),
              pltpu.make_async_copy(dst_hbm.at[eoff(i)], dstv.at[ib],
                                    sidx.at[ib]))

    def idx_start(i):
      a, b = idx_copies(i)
      a.start()
      b.start()

    def idx_wait(i):
      a, b = idx_copies(i)
      a.wait()
      b.wait()

    def gather_copy(i):
      return pltpu.make_async_copy(g_hbm.at[srcv.at[lax.rem(i, 4)]],
                                   rows.at[lax.rem(i, 3)],
                                   sgat.at[lax.rem(i, 3)])

    def scat_start(i):
      pltpu.async_copy(rows.at[lax.rem(i, 3)], acc.at[dstv.at[lax.rem(i, 4)]],
                       sscat.at[lax.rem(i, 2)], add=True)

    def scat_wait(i):
      pltpu.make_async_copy(rows.at[lax.rem(i, 3)],
                            acc.at[dstv.at[lax.rem(i, 4)]],
                            sscat.at[lax.rem(i, 2)]).wait()

    idx_start(0)

    @pl.when(1 < ti)
    def _():
      idx_start(1)

    @pl.when(2 < ti)
    def _():
      idx_start(2)

    idx_wait(0)
    gather_copy(0).start()

    @pl.when(1 < ti)
    def _():
      idx_wait(1)
      gather_copy(1).start()

    plsc.subcore_barrier()

    @pl.loop(0, ITERS)
    def _(i):
      @pl.when(i < ti)
      def _():
        @pl.when(i >= 1)
        def _():
          scat_wait(i - 1)

        @pl.when(i + 3 < ti)
        def _():
          idx_start(i + 3)

        @pl.when(i + 2 < ti)
        def _():
          idx_wait(i + 2)
          gather_copy(i + 2).start()

        gather_copy(i).wait()
        scat_start(i)

    scat_wait(ti - 1)
    plsc.subcore_barrier()

    def writeback(start, nrows):
      pltpu.sync_copy(acc.at[pl.ds(start, nrows)],
                      out_hbm.at[pl.ds(pl.multiple_of(cid * N + start, 8),
                                       nrows)])

    _rowwise(writeback, sid)

  return k(g, src, dst, zeros)


_DOT = functools.partial(jnp.dot, preferred_element_type=jnp.float32,
                         precision=lax.Precision.HIGHEST)


def _tc_call(body, n_in, extra_specs):
  return pl.pallas_call(
      body,
      grid=(N // BN,),
      in_specs=extra_specs,
      out_specs=pl.BlockSpec((BN, D), lambda i: (i, 0)),
      out_shape=jax.ShapeDtypeStruct((N, D), jnp.float32),
  )


def _affine_scaled(x, WT, b, dinv):
  def body(x_ref, w_ref, b_ref, s_ref, o_ref):
    o_ref[...] = (_DOT(x_ref[...], w_ref[...]) + b_ref[...]) * s_ref[...]

  return pl.pallas_call(
      body,
      grid=(N // BN,),
      in_specs=[pl.BlockSpec((BN, D), lambda i: (i, 0)),
                pl.BlockSpec((D, D), lambda i: (0, 0)),
                pl.BlockSpec((1, D), lambda i: (0, 0)),
                pl.BlockSpec((BN, 1), lambda i: (i, 0))],
      out_specs=pl.BlockSpec((BN, D), lambda i: (i, 0)),
      out_shape=jax.ShapeDtypeStruct((N, D), jnp.float32),
  )(x, WT, b.reshape(1, D), dinv)


def _combine_linear(acc, dinv, b, WT):
  def body(a0_ref, a1_ref, s_ref, b_ref, w_ref, o_ref):
    h = jnp.maximum((a0_ref[...] + a1_ref[...]) * s_ref[...] + b_ref[...], 0.0)
    o_ref[...] = _DOT(h, w_ref[...]) * s_ref[...]

  nb = N // BN
  return pl.pallas_call(
      body,
      grid=(nb,),
      in_specs=[pl.BlockSpec((BN, D), lambda i: (i, 0)),
                pl.BlockSpec((BN, D), lambda i: (i + nb, 0)),
                pl.BlockSpec((BN, 1), lambda i: (i, 0)),
                pl.BlockSpec((1, D), lambda i: (0, 0)),
                pl.BlockSpec((D, D), lambda i: (0, 0))],
      out_specs=pl.BlockSpec((BN, D), lambda i: (i, 0)),
      out_shape=jax.ShapeDtypeStruct((N, D), jnp.float32),
  )(acc, acc, dinv, b.reshape(1, D), WT)


def _finalize(acc, dinv, b):
  def body(a0_ref, a1_ref, s_ref, b_ref, o_ref):
    o_ref[...] = jnp.maximum(
        (a0_ref[...] + a1_ref[...]) * s_ref[...] + b_ref[...], 0.0)

  nb = N // BN
  return pl.pallas_call(
      body,
      grid=(nb,),
      in_specs=[pl.BlockSpec((BN, D), lambda i: (i, 0)),
                pl.BlockSpec((BN, D), lambda i: (i + nb, 0)),
                pl.BlockSpec((BN, 1), lambda i: (i, 0)),
                pl.BlockSpec((1, D), lambda i: (0, 0))],
      out_specs=pl.BlockSpec((BN, D), lambda i: (i, 0)),
      out_shape=jax.ShapeDtypeStruct((N, D), jnp.float32),
  )(acc, acc, dinv, b.reshape(1, D))


def kernel(x, edge_index, W_in, b_in, W1, b1, W2, b2):
  src = edge_index[0]
  dst = edge_index[1]
  zeros_nd = jnp.zeros((N, D), jnp.float32)

  deg2 = _sc_degree(dst)
  deg = deg2.reshape(NW, N).sum(axis=0) + 1.0
  dinv = lax.rsqrt(deg).reshape(N, 1)

  Wc = W_in.T @ W1.T
  bc = b_in @ W1.T
  g1 = _affine_scaled(x, Wc, bc, dinv)
  acc1 = _sc_scatter(g1, src, dst, zeros_nd)
  g2 = _combine_linear(acc1, dinv, b1, W2.T)
  acc2 = _sc_scatter(g2, src, dst, zeros_nd)
  return _finalize(acc2, dinv, b2)


# speedup vs baseline: 35.2367x; 1.0496x over previous
import dataclasses
import functools

import jax
import jax.numpy as jnp
from jax import lax
from jax.experimental import pallas as pl
from jax.experimental.pallas import tpu as pltpu
from jax.experimental.pallas import tpu_sc as plsc

N = 10000
E = 320000
D = 128

NC = 2
NS = 16
NW = NC * NS
CH = 128
NCHUNK = E // CH
ITERS = -(-NCHUNK // NW)
ROWS_PT = 624
ROWS_LAST = N - (NS - 1) * ROWS_PT

DEG_BLK = 2000
DEG_NB = E // NW // DEG_BLK


def _rowwise(copy_fn, sid):
  base = pl.multiple_of(sid * ROWS_PT, 8)

  @pl.when(sid < NS - 1)
  def _():
    copy_fn(base, ROWS_PT)

  @pl.when(sid == NS - 1)
  def _():
    copy_fn(base, ROWS_LAST)

BN = 512

def _sc_compiler_params():
  cp = pltpu.CompilerParams()
  if "needs_layout_passes" in pltpu.CompilerParams.__dataclass_fields__:
    cp = dataclasses.replace(cp, needs_layout_passes=False)
  return cp


@functools.cache
def _mesh():
  return plsc.VectorSubcoreMesh(core_axis_name="c", subcore_axis_name="s",
                                num_cores=NC, num_subcores=NS)


def _sc_degree(dst):
  @functools.partial(
      pl.kernel,
      out_type=jax.ShapeDtypeStruct((NW * N,), jnp.float32),
      mesh=_mesh(),
      scratch_types=[
          pltpu.VMEM((2 * DEG_BLK,), jnp.int32),
          pltpu.VMEM((N,), jnp.float32),
          pltpu.SemaphoreType.DMA((2,)),
      ],
      compiler_params=_sc_compiler_params(),
  )
  def k(dst_hbm, out_hbm, idxv, hist, sidx):
    cid = lax.axis_index("c")
    sid = lax.axis_index("s")
    w = cid * NS + sid

    @pl.loop(0, N // 16)
    def _(i):
      hist[pl.ds(i * 16, 16)] = jnp.zeros((16,), jnp.float32)

    def idx_copy(i):
      ib = lax.rem(i, 2)
      off = pl.multiple_of(w * (E // NW) + i * DEG_BLK, 8)
      return pltpu.make_async_copy(dst_hbm.at[pl.ds(off, DEG_BLK)],
                                   idxv.at[pl.ds(ib * DEG_BLK, DEG_BLK)],
                                   sidx.at[ib])

    idx_copy(0).start()

    @pl.loop(0, DEG_NB)
    def _(i):
      idx_copy(i).wait()

      @pl.when(i + 1 < DEG_NB)
      def _():
        idx_copy(i + 1).start()

      base = lax.rem(i, 2) * DEG_BLK

      @pl.loop(0, DEG_BLK // 16)
      def _(j):
        v = idxv[pl.ds(base + j * 16, 16)]
        plsc.addupdate_scatter(hist, [v], jnp.ones((16,), jnp.float32))

    pltpu.sync_copy(hist, out_hbm.at[pl.ds(pl.multiple_of(w * N, 8), N)])

  return k(dst)


def _sc_scatter(g, src, dst, zeros):
  @functools.partial(
      pl.kernel,
      out_type=[jax.ShapeDtypeStruct((N, D), jnp.float32),
                jax.ShapeDtypeStruct((N, D), jnp.float32)],
      mesh=_mesh(),
      scratch_types=[
          pltpu.VMEM((4, CH), jnp.int32),
          pltpu.VMEM((4, CH), jnp.int32),
          pltpu.VMEM((3, CH, D), jnp.float32),
          pltpu.VMEM_SHARED((N, D), jnp.float32),
          pltpu.SemaphoreType.DMA((4,)),
          pltpu.SemaphoreType.DMA((3,)),
          pltpu.SemaphoreType.DMA((2,)),
      ],
  )
  def k(g_hbm, src_hbm, dst_hbm, zeros_hbm, out0_hbm, out1_hbm, srcv, dstv,
        rows, acc, sidx, sgat, sscat):
    cid = lax.axis_index("c")
    sid = lax.axis_index("s")
    w = cid * NS + sid
    ti = (NCHUNK - 1 - w) // NW + 1

    def init(start, nrows):
      s = pl.ds(start, nrows)

      @pl.when(cid == 0)
      def _():
        pltpu.sync_copy(g_hbm.at[s], acc.at[s])

      @pl.when(cid != 0)
      def _():
        pltpu.sync_copy(zeros_hbm.at[s], acc.at[s])

    _rowwise(init, sid)

    def eoff(i):
      return pl.ds(pl.multiple_of((w + i * NW) * CH, 8), CH)

    def idx_copies(i):
      ib = lax.rem(i, 4)
      return (pltpu.make_async_copy(src_hbm.at[eoff(i)], srcv.at[ib],
                                    sidx.at[ib]),
              pltpu.make_async_copy(dst_hbm.at[eoff(i)], dstv.at[ib],
                                    sidx.at[ib]))

    def idx_start(i):
      a, b = idx_copies(i)
      a.start()
      b.start()

    def idx_wait(i):
      a, b = idx_copies(i)
      a.wait()
      b.wait()

    def gather_copy(i):
      return pltpu.make_async_copy(g_hbm.at[srcv.at[lax.rem(i, 4)]],
                                   rows.at[lax.rem(i, 3)],
                                   sgat.at[lax.rem(i, 3)])

    def scat_start(i):
      pltpu.async_copy(rows.at[lax.rem(i, 3)], acc.at[dstv.at[lax.rem(i, 4)]],
                       sscat.at[lax.rem(i, 2)], add=True)

    def scat_wait(i):
      pltpu.make_async_copy(rows.at[lax.rem(i, 3)],
                            acc.at[dstv.at[lax.rem(i, 4)]],
                            sscat.at[lax.rem(i, 2)]).wait()

    idx_start(0)

    @pl.when(1 < ti)
    def _():
      idx_start(1)

    @pl.when(2 < ti)
    def _():
      idx_start(2)

    idx_wait(0)
    gather_copy(0).start()

    @pl.when(1 < ti)
    def _():
      idx_wait(1)
      gather_copy(1).start()

    plsc.subcore_barrier()

    @pl.loop(0, ITERS)
    def _(i):
      @pl.when(i < ti)
      def _():
        @pl.when(i >= 1)
        def _():
          scat_wait(i - 1)

        @pl.when(i + 3 < ti)
        def _():
          idx_start(i + 3)

        @pl.when(i + 2 < ti)
        def _():
          idx_wait(i + 2)
          gather_copy(i + 2).start()

        gather_copy(i).wait()
        scat_start(i)

    scat_wait(ti - 1)
    plsc.subcore_barrier()

    def writeback(start, nrows):
      s = pl.ds(start, nrows)

      @pl.when(cid == 0)
      def _():
        pltpu.sync_copy(acc.at[s], out0_hbm.at[s])

      @pl.when(cid != 0)
      def _():
        pltpu.sync_copy(acc.at[s], out1_hbm.at[s])

    _rowwise(writeback, sid)

  return k(g, src, dst, zeros)


_DOT = functools.partial(jnp.dot, preferred_element_type=jnp.float32,
                         precision=lax.Precision.HIGHEST)


def _tc_call(body, n_in, extra_specs):
  return pl.pallas_call(
      body,
      grid=(N // BN,),
      in_specs=extra_specs,
      out_specs=pl.BlockSpec((BN, D), lambda i: (i, 0)),
      out_shape=jax.ShapeDtypeStruct((N, D), jnp.float32),
  )


def _dinv_of(deg_blk):
  return lax.rsqrt(jnp.sum(deg_blk, axis=0) + 1.0)[:, None]


_GRID = (pl.cdiv(N, BN),)
_BLK = pl.BlockSpec((BN, D), lambda i: (i, 0))
_WBLK = pl.BlockSpec((D, D), lambda i: (0, 0))
_BBLK = pl.BlockSpec((1, D), lambda i: (0, 0))
_DEGBLK = pl.BlockSpec((NW, BN), lambda i: (0, i))


def _affine_scaled(x, WT, b, deg2d):
  def body(x_ref, w_ref, b_ref, g_ref, o_ref):
    s = _dinv_of(g_ref[...])
    o_ref[...] = (_DOT(x_ref[...], w_ref[...]) + b_ref[...]) * s

  return pl.pallas_call(
      body,
      grid=_GRID,
      in_specs=[_BLK, _WBLK, _BBLK, _DEGBLK],
      out_specs=_BLK,
      out_shape=jax.ShapeDtypeStruct((N, D), jnp.float32),
  )(x, WT, b.reshape(1, D), deg2d)


def _combine_linear(a0, a1, deg2d, b, WT):
  def body(a0_ref, a1_ref, g_ref, b_ref, w_ref, o_ref):
    s = _dinv_of(g_ref[...])
    h = jnp.maximum((a0_ref[...] + a1_ref[...]) * s + b_ref[...], 0.0)
    o_ref[...] = _DOT(h, w_ref[...]) * s

  return pl.pallas_call(
      body,
      grid=_GRID,
      in_specs=[_BLK, _BLK, _DEGBLK, _BBLK, _WBLK],
      out_specs=_BLK,
      out_shape=jax.ShapeDtypeStruct((N, D), jnp.float32),
  )(a0, a1, deg2d, b.reshape(1, D), WT)


def _finalize(a0, a1, deg2d, b):
  def body(a0_ref, a1_ref, g_ref, b_ref, o_ref):
    s = _dinv_of(g_ref[...])
    o_ref[...] = jnp.maximum(
        (a0_ref[...] + a1_ref[...]) * s + b_ref[...], 0.0)

  return pl.pallas_call(
      body,
      grid=_GRID,
      in_specs=[_BLK, _BLK, _DEGBLK, _BBLK],
      out_specs=_BLK,
      out_shape=jax.ShapeDtypeStruct((N, D), jnp.float32),
  )(a0, a1, deg2d, b.reshape(1, D))


def kernel(x, edge_index, W_in, b_in, W1, b1, W2, b2):
  src = edge_index[0]
  dst = edge_index[1]
  zeros_nd = jnp.zeros((N, D), jnp.float32)

  deg2d = _sc_degree(dst).reshape(NW, N)

  Wc = W_in.T @ W1.T
  bc = b_in @ W1.T
  g1 = _affine_scaled(x, Wc, bc, deg2d)
  a0, a1 = _sc_scatter(g1, src, dst, zeros_nd)
  g2 = _combine_linear(a0, a1, deg2d, b1, W2.T)
  a0, a1 = _sc_scatter(g2, src, dst, zeros_nd)
  return _finalize(a0, a1, deg2d, b2)


# speedup vs baseline: 35.6575x; 1.0119x over previous
import dataclasses
import functools

import jax
import jax.numpy as jnp
from jax import lax
from jax.experimental import pallas as pl
from jax.experimental.pallas import tpu as pltpu
from jax.experimental.pallas import tpu_sc as plsc

N = 10000
E = 320000
D = 128

NC = 2
NS = 16
NW = NC * NS
CH = 128
NCHUNK = E // CH
ITERS = -(-NCHUNK // NW)
ROWS_PT = 624
ROWS_LAST = N - (NS - 1) * ROWS_PT

DEG_BLK = 2000
DEG_NB = E // NW // DEG_BLK


def _rowwise(copy_fn, sid):
  base = pl.multiple_of(sid * ROWS_PT, 8)

  @pl.when(sid < NS - 1)
  def _():
    copy_fn(base, ROWS_PT)

  @pl.when(sid == NS - 1)
  def _():
    copy_fn(base, ROWS_LAST)

BN = 512

def _sc_compiler_params():
  cp = pltpu.CompilerParams()
  if "needs_layout_passes" in pltpu.CompilerParams.__dataclass_fields__:
    cp = dataclasses.replace(cp, needs_layout_passes=False)
  return cp


@functools.cache
def _mesh():
  return plsc.VectorSubcoreMesh(core_axis_name="c", subcore_axis_name="s",
                                num_cores=NC, num_subcores=NS)


def _sc_degree(dst):
  @functools.partial(
      pl.kernel,
      out_type=jax.ShapeDtypeStruct((NW * N,), jnp.float32),
      mesh=_mesh(),
      scratch_types=[
          pltpu.VMEM((2 * DEG_BLK,), jnp.int32),
          pltpu.VMEM((N,), jnp.float32),
          pltpu.SemaphoreType.DMA((2,)),
      ],
      compiler_params=_sc_compiler_params(),
  )
  def k(dst_hbm, out_hbm, idxv, hist, sidx):
    cid = lax.axis_index("c")
    sid = lax.axis_index("s")
    w = cid * NS + sid

    @pl.loop(0, N // 16)
    def _(i):
      hist[pl.ds(i * 16, 16)] = jnp.zeros((16,), jnp.float32)

    def idx_copy(i):
      ib = lax.rem(i, 2)
      off = pl.multiple_of(w * (E // NW) + i * DEG_BLK, 8)
      return pltpu.make_async_copy(dst_hbm.at[pl.ds(off, DEG_BLK)],
                                   idxv.at[pl.ds(ib * DEG_BLK, DEG_BLK)],
                                   sidx.at[ib])

    idx_copy(0).start()

    @pl.loop(0, DEG_NB)
    def _(i):
      idx_copy(i).wait()

      @pl.when(i + 1 < DEG_NB)
      def _():
        idx_copy(i + 1).start()

      base = lax.rem(i, 2) * DEG_BLK

      @pl.loop(0, DEG_BLK // 16)
      def _(j):
        v = idxv[pl.ds(base + j * 16, 16)]
        plsc.addupdate_scatter(hist, [v], jnp.ones((16,), jnp.float32))

    pltpu.sync_copy(hist, out_hbm.at[pl.ds(pl.multiple_of(w * N, 8), N)])

  return k(dst)


def _sc_scatter(g, src, dst, zeros):
  @functools.partial(
      pl.kernel,
      out_type=[jax.ShapeDtypeStruct((N, D), jnp.float32),
                jax.ShapeDtypeStruct((N, D), jnp.float32)],
      mesh=_mesh(),
      scratch_types=[
          pltpu.VMEM((4, CH), jnp.int32),
          pltpu.VMEM((4, CH), jnp.int32),
          pltpu.VMEM((3, CH, D), jnp.float32),
          pltpu.VMEM_SHARED((N, D), jnp.float32),
          pltpu.SemaphoreType.DMA((4,)),
          pltpu.SemaphoreType.DMA((3,)),
          pltpu.SemaphoreType.DMA((2,)),
      ],
  )
  def k(g_hbm, src_hbm, dst_hbm, zeros_hbm, out0_hbm, out1_hbm, srcv, dstv,
        rows, acc, sidx, sgat, sscat):
    cid = lax.axis_index("c")
    sid = lax.axis_index("s")
    w = cid * NS + sid
    ti = (NCHUNK - 1 - w) // NW + 1

    def init(start, nrows):
      s = pl.ds(start, nrows)

      @pl.when(cid == 0)
      def _():
        pltpu.sync_copy(g_hbm.at[s], acc.at[s])

      @pl.when(cid != 0)
      def _():
        pltpu.sync_copy(zeros_hbm.at[s], acc.at[s])

    _rowwise(init, sid)

    def eoff(i):
      return pl.ds(pl.multiple_of((w + i * NW) * CH, 8), CH)

    def idx_copies(i):
      ib = lax.rem(i, 4)
      return (pltpu.make_async_copy(src_hbm.at[eoff(i)], srcv.at[ib],
                                    sidx.at[ib]),
              pltpu.make_async_copy(dst_hbm.at[eoff(i)], dstv.at[ib],
                                    sidx.at[ib]))

    def idx_start(i):
      a, b = idx_copies(i)
      a.start()
      b.start()

    def idx_wait(i):
      a, b = idx_copies(i)
      a.wait()
      b.wait()

    def gather_copy(i):
      return pltpu.make_async_copy(g_hbm.at[srcv.at[lax.rem(i, 4)]],
                                   rows.at[lax.rem(i, 3)],
                                   sgat.at[lax.rem(i, 3)])

    def scat_start(i):
      pltpu.async_copy(rows.at[lax.rem(i, 3)], acc.at[dstv.at[lax.rem(i, 4)]],
                       sscat.at[lax.rem(i, 2)], add=True)

    def scat_wait(i):
      pltpu.make_async_copy(rows.at[lax.rem(i, 3)],
                            acc.at[dstv.at[lax.rem(i, 4)]],
                            sscat.at[lax.rem(i, 2)]).wait()

    idx_start(0)

    @pl.when(1 < ti)
    def _():
      idx_start(1)

    @pl.when(2 < ti)
    def _():
      idx_start(2)

    idx_wait(0)
    gather_copy(0).start()

    @pl.when(1 < ti)
    def _():
      idx_wait(1)
      gather_copy(1).start()

    plsc.subcore_barrier()

    @pl.loop(0, ITERS)
    def _(i):
      @pl.when(i < ti)
      def _():
        @pl.when(i >= 1)
        def _():
          scat_wait(i - 1)

        @pl.when(i + 3 < ti)
        def _():
          idx_start(i + 3)

        @pl.when(i + 2 < ti)
        def _():
          idx_wait(i + 2)
          gather_copy(i + 2).start()

        gather_copy(i).wait()
        scat_start(i)

    scat_wait(ti - 1)
    plsc.subcore_barrier()

    def writeback(start, nrows):
      s = pl.ds(start, nrows)

      @pl.when(cid == 0)
      def _():
        pltpu.sync_copy(acc.at[s], out0_hbm.at[s])

      @pl.when(cid != 0)
      def _():
        pltpu.sync_copy(acc.at[s], out1_hbm.at[s])

    _rowwise(writeback, sid)

  return k(g, src, dst, zeros)


_DOT = functools.partial(jnp.dot, preferred_element_type=jnp.float32,
                         precision=lax.Precision.DEFAULT)


def _dinv_of(deg_blk):
  return lax.rsqrt(jnp.sum(deg_blk, axis=0) + 1.0)[:, None]


_GRID = (pl.cdiv(N, BN),)
_BLK = pl.BlockSpec((BN, D), lambda i: (i, 0))
_WBLK = pl.BlockSpec((D, D), lambda i: (0, 0))
_BBLK = pl.BlockSpec((1, D), lambda i: (0, 0))
_DEGBLK = pl.BlockSpec((NW, BN), lambda i: (0, i))


def _affine_scaled(x, WT, b, deg2d):
  def body(x_ref, w_ref, b_ref, g_ref, o_ref):
    s = _dinv_of(g_ref[...])
    o_ref[...] = (_DOT(x_ref[...], w_ref[...]) + b_ref[...]) * s

  return pl.pallas_call(
      body,
      grid=_GRID,
      in_specs=[_BLK, _WBLK, _BBLK, _DEGBLK],
      out_specs=_BLK,
      out_shape=jax.ShapeDtypeStruct((N, D), jnp.float32),
  )(x, WT, b.reshape(1, D), deg2d)


def _combine_linear(a0, a1, deg2d, b, WT):
  def body(a0_ref, a1_ref, g_ref, b_ref, w_ref, o_ref):
    s = _dinv_of(g_ref[...])
    h = jnp.maximum((a0_ref[...] + a1_ref[...]) * s + b_ref[...], 0.0)
    o_ref[...] = _DOT(h, w_ref[...]) * s

  return pl.pallas_call(
      body,
      grid=_GRID,
      in_specs=[_BLK, _BLK, _DEGBLK, _BBLK, _WBLK],
      out_specs=_BLK,
      out_shape=jax.ShapeDtypeStruct((N, D), jnp.float32),
  )(a0, a1, deg2d, b.reshape(1, D), WT)


def _finalize(a0, a1, deg2d, b):
  def body(a0_ref, a1_ref, g_ref, b_ref, o_ref):
    s = _dinv_of(g_ref[...])
    o_ref[...] = jnp.maximum(
        (a0_ref[...] + a1_ref[...]) * s + b_ref[...], 0.0)

  return pl.pallas_call(
      body,
      grid=_GRID,
      in_specs=[_BLK, _BLK, _DEGBLK, _BBLK],
      out_specs=_BLK,
      out_shape=jax.ShapeDtypeStruct((N, D), jnp.float32),
  )(a0, a1, deg2d, b.reshape(1, D))


def kernel(x, edge_index, W_in, b_in, W1, b1, W2, b2):
  src = edge_index[0]
  dst = edge_index[1]
  zeros_nd = jnp.zeros((N, D), jnp.float32)

  deg2d = _sc_degree(dst).reshape(NW, N)

  Wc = W_in.T @ W1.T
  bc = b_in @ W1.T
  g1 = _affine_scaled(x, Wc, bc, deg2d)
  a0, a1 = _sc_scatter(g1, src, dst, zeros_nd)
  g2 = _combine_linear(a0, a1, deg2d, b1, W2.T)
  a0, a1 = _sc_scatter(g2, src, dst, zeros_nd)
  return _finalize(a0, a1, deg2d, b2)
